# baseline (speedup 1.0000x reference)
import functools

import jax
import jax.numpy as jnp
from jax import lax
from jax.experimental import pallas as pl
from jax.experimental.pallas import tpu as pltpu
from jax.experimental.pallas import tpu_sc as plsc

_B = 4
_L = 8192
_EMB = 64
_NC = 2
_NS = 16
_NW = _NC * _NS
_CHUNK = _L // _NW
_GCH = 128
_NG = _CHUNK // _GCH
_PACK = 32


def _sc_embed(idx2d, token_table, pos_table):
    mesh = plsc.VectorSubcoreMesh(core_axis_name="c", subcore_axis_name="s")

    @functools.partial(
        pl.kernel,
        mesh=mesh,
        out_type=jax.ShapeDtypeStruct((_B * _L, _EMB), jnp.bfloat16),
        scratch_types=[
            pltpu.VMEM((_NG, _GCH), jnp.int32),
            pltpu.VMEM((_GCH, _EMB), jnp.bfloat16),
            pltpu.VMEM((_CHUNK, _EMB), jnp.bfloat16),
            pltpu.SemaphoreType.DMA,
        ],
        compiler_params=pltpu.CompilerParams(use_tc_tiling_on_sc=False),
    )
    def k(idx_hbm, tok_hbm, pos_hbm, out_hbm, idx_v, rows_v, pos_v, sem):
        c = lax.axis_index("c")
        s = lax.axis_index("s")
        w = s * _NC + c
        l0 = w * _CHUNK
        pltpu.sync_copy(pos_hbm.at[pl.ds(l0, _CHUNK)], pos_v)
        for b in range(_B):
            cid0 = b * (_L // _GCH) + w * _NG
            pltpu.sync_copy(idx_hbm.at[pl.ds(cid0, _NG)], idx_v)
            for h in range(_NG):
                pltpu.async_copy(tok_hbm.at[idx_v.at[h]], rows_v, sem).wait()

                def add_body(r, _, h=h):
                    for j in range(_EMB // _PACK):
                        sl = pl.ds(j * _PACK, _PACK)
                        rows_v[r, sl] = rows_v[r, sl] + pos_v[h * _GCH + r, sl]
                    return 0

                lax.fori_loop(0, _GCH, add_body, 0)
                row0 = b * _L + l0 + h * _GCH
                pltpu.sync_copy(rows_v, out_hbm.at[pl.ds(row0, _GCH)])

    return k(idx2d, token_table, pos_table)


def kernel(inputs, token_table, pos_table):
    idx2d = inputs.reshape(_B * _L // _GCH, _GCH).astype(jnp.int32)
    tok_bf = token_table.astype(jnp.bfloat16)
    pos_bf = pos_table.astype(jnp.bfloat16)
    out = _sc_embed(idx2d, tok_bf, pos_bf)
    return out.astype(jnp.float32).reshape(_B, _L, _EMB)


# speedup vs baseline: 1.2995x; 1.2995x over previous
import functools

import jax
import jax.numpy as jnp
from jax import lax
from jax.experimental import pallas as pl
from jax.experimental.pallas import tpu as pltpu
from jax.experimental.pallas import tpu_sc as plsc

_B = 4
_L = 8192
_EMB = 64
_NC = 2
_NS = 16
_NW = _NC * _NS
_CHUNK = _L // _NW
_GCH = 128
_NG = _CHUNK // _GCH
_LANES = 16


def _sc_embed(idx2d, mask, tok2, pos_table):
    mesh = plsc.VectorSubcoreMesh(core_axis_name="c", subcore_axis_name="s")

    @functools.partial(
        pl.kernel,
        mesh=mesh,
        out_type=jax.ShapeDtypeStruct((_B * _L, _EMB), jnp.float32),
        scratch_types=[
            pltpu.VMEM((_NG, _GCH), jnp.int32),
            pltpu.VMEM((_GCH, 2 * _EMB), jnp.float32),
            pltpu.VMEM((_GCH, _EMB), jnp.float32),
            pltpu.VMEM((_GCH, _EMB), jnp.float32),
            pltpu.VMEM((_CHUNK, _EMB), jnp.float32),
            pltpu.SemaphoreType.DMA,
        ],
    )
    def k(idx_hbm, mask_hbm, tok_hbm, pos_hbm, out_hbm,
          idx_v, rows_v, mask_v, res_v, pos_v, sem):
        c = lax.axis_index("c")
        s = lax.axis_index("s")
        w = s * _NC + c
        l0 = w * _CHUNK
        pltpu.sync_copy(pos_hbm.at[pl.ds(l0, _CHUNK)], pos_v)
        for b in range(_B):
            cid0 = b * (_L // _GCH) + w * _NG
            pltpu.sync_copy(idx_hbm.at[pl.ds(cid0, _NG)], idx_v)
            for h in range(_NG):
                row0 = b * _L + l0 + h * _GCH
                pltpu.sync_copy(mask_hbm.at[pl.ds(row0, _GCH)], mask_v)
                pltpu.async_copy(tok_hbm.at[idx_v.at[h]], rows_v, sem).wait()

                def body(r, _, h=h):
                    for j in range(_EMB // _LANES):
                        sl = pl.ds(j * _LANES, _LANES)
                        lo = rows_v[r, sl]
                        hi = rows_v[r, pl.ds(_EMB + j * _LANES, _LANES)]
                        m = mask_v[r, sl]
                        p = pos_v[h * _GCH + r, sl]
                        res_v[r, sl] = lo + (hi - lo) * m + p
                    return 0

                lax.fori_loop(0, _GCH, body, 0)
                pltpu.sync_copy(res_v, out_hbm.at[pl.ds(row0, _GCH)])

    return k(idx2d, mask, tok2, pos_table)


def kernel(inputs, token_table, pos_table):
    flat = inputs.reshape(_B * _L).astype(jnp.int32)
    idx2d = (flat >> 1).reshape(_B * _L // _GCH, _GCH)
    mask = jnp.broadcast_to(
        (flat & 1).astype(jnp.float32)[:, None], (_B * _L, _EMB)
    )
    tok2 = token_table.reshape(1000000 // 2, 2 * _EMB)
    out = _sc_embed(idx2d, mask, tok2, pos_table)
    return out.reshape(_B, _L, _EMB)


# speedup vs baseline: 1.3606x; 1.0470x over previous
import functools

import jax
import jax.numpy as jnp
from jax import lax
from jax.experimental import pallas as pl
from jax.experimental.pallas import tpu as pltpu
from jax.experimental.pallas import tpu_sc as plsc

_B = 4
_L = 8192
_EMB = 64
_NC = 2
_NS = 16
_NW = _NC * _NS
_ROWS = _B * _L
_CHUNK = _ROWS // _NW
_GCH = 128
_NG = _CHUNK // _GCH


def _sc_gather(idx2d, token_table):
    mesh = plsc.VectorSubcoreMesh(core_axis_name="c", subcore_axis_name="s")

    @functools.partial(
        pl.kernel,
        mesh=mesh,
        out_type=jax.ShapeDtypeStruct((_ROWS, _EMB), jnp.float32),
        scratch_types=[
            pltpu.VMEM((_NG, _GCH), jnp.int32),
            pltpu.VMEM((_CHUNK, _EMB), jnp.float32),
            pltpu.SemaphoreType.DMA,
        ],
        compiler_params=pltpu.CompilerParams(use_tc_tiling_on_sc=False),
    )
    def k(idx_hbm, tok_hbm, out_hbm, idx_v, rows_v, sem):
        c = lax.axis_index("c")
        s = lax.axis_index("s")
        w = s * _NC + c
        pltpu.sync_copy(idx_hbm.at[pl.ds(w * _NG, _NG)], idx_v)
        cps = []
        for h in range(_NG):
            cps.append(
                pltpu.async_copy(
                    tok_hbm.at[idx_v.at[h]],
                    rows_v.at[pl.ds(h * _GCH, _GCH)],
                    sem,
                )
            )
        for cp in cps:
            cp.wait()
        pltpu.sync_copy(rows_v, out_hbm.at[pl.ds(w * _CHUNK, _CHUNK)])

    return k(idx2d, token_table)


def kernel(inputs, token_table, pos_table):
    idx2d = inputs.reshape(_ROWS // _GCH, _GCH).astype(jnp.int32)
    gathered = _sc_gather(idx2d, token_table)
    return gathered.reshape(_B, _L, _EMB) + pos_table[None, :, :]
